# Initial kernel scaffold; baseline (speedup 1.0000x reference)
import functools

import jax
import jax.numpy as jnp
from jax import lax
from jax.experimental import pallas as pl
from jax.experimental.pallas import tpu as pltpu
from jax.experimental.pallas import tpu_sc as plsc

BATCH = 16384
HIST = 200
EMB_DIM = 32

NUM_WORKERS = 32
B_TOTAL = BATCH * HIST
B_PER_W = B_TOTAL // NUM_WORKERS
CHUNK = 1024
N_CHUNKS = B_PER_W // CHUNK


def _emb_kernel(idx_hbm, table_hbm, out_hbm, idx_v, rows_v, sem):
    wid = lax.axis_index("s") * 2 + lax.axis_index("c")
    base = wid * B_PER_W

    def body(i, _):
        off = base + i * CHUNK
        pltpu.sync_copy(idx_hbm.at[pl.ds(off, CHUNK)], idx_v)
        pltpu.async_copy(table_hbm.at[idx_v], rows_v, sem).wait()
        pltpu.sync_copy(rows_v, out_hbm.at[pl.ds(off, CHUNK)])
        return ()

    lax.fori_loop(0, N_CHUNKS, body, ())


@jax.jit
def _embedding_lookup(x_flat, table):
    mesh = plsc.VectorSubcoreMesh(core_axis_name="c", subcore_axis_name="s")
    k = functools.partial(
        pl.kernel,
        out_type=jax.ShapeDtypeStruct((B_TOTAL, EMB_DIM), jnp.float32),
        mesh=mesh,
        scratch_types=[
            pltpu.VMEM((CHUNK,), jnp.int32),
            pltpu.VMEM((CHUNK, EMB_DIM), jnp.float32),
            pltpu.SemaphoreType.DMA,
        ],
    )(_emb_kernel)
    return k(x_flat, table)


def kernel(x, table):
    out = _embedding_lookup(x.reshape(-1).astype(jnp.int32), table)
    return out.reshape(BATCH, HIST, EMB_DIM)


# speedup vs baseline: 4.8020x; 4.8020x over previous
import functools

import jax
import jax.numpy as jnp
from jax import lax
from jax.experimental import pallas as pl
from jax.experimental.pallas import tpu as pltpu
from jax.experimental.pallas import tpu_sc as plsc

BATCH = 16384
HIST = 200
EMB_DIM = 32

NUM_WORKERS = 32
B_TOTAL = BATCH * HIST
B_PER_W = B_TOTAL // NUM_WORKERS
CHUNK = 1024
N_CHUNKS = B_PER_W // CHUNK


def _emb_kernel(idx_hbm, table_hbm, out_hbm, idx_v, rows_v, sem):
    wid = lax.axis_index("s") * 2 + lax.axis_index("c")
    base = wid * B_PER_W

    def body(i, _):
        off = base + i * CHUNK
        pltpu.sync_copy(idx_hbm.at[pl.ds(off, CHUNK)], idx_v)
        pltpu.async_copy(table_hbm.at[idx_v], rows_v, sem).wait()
        pltpu.sync_copy(rows_v, out_hbm.at[pl.ds(off, CHUNK)])
        return ()

    lax.fori_loop(0, N_CHUNKS, body, ())


@jax.jit
def _embedding_lookup(x_flat, table):
    mesh = plsc.VectorSubcoreMesh(core_axis_name="c", subcore_axis_name="s")
    k = functools.partial(
        pl.kernel,
        out_type=jax.ShapeDtypeStruct((B_TOTAL, EMB_DIM), jnp.float32),
        mesh=mesh,
        scratch_types=[
            pltpu.VMEM((CHUNK,), jnp.int32),
            pltpu.VMEM((CHUNK, EMB_DIM), jnp.float32),
            pltpu.SemaphoreType.DMA,
        ],
        compiler_params=pltpu.CompilerParams(use_tc_tiling_on_sc=False),
    )(_emb_kernel)
    return k(x_flat, table)


def kernel(x, table):
    out = _embedding_lookup(x.reshape(-1).astype(jnp.int32), table)
    return out.reshape(BATCH, HIST, EMB_DIM)


# speedup vs baseline: 5.0416x; 1.0499x over previous
import functools

import jax
import jax.numpy as jnp
from jax import lax
from jax.experimental import pallas as pl
from jax.experimental.pallas import tpu as pltpu
from jax.experimental.pallas import tpu_sc as plsc

BATCH = 16384
HIST = 200
EMB_DIM = 32

NUM_WORKERS = 32
B_TOTAL = BATCH * HIST
B_PER_W = B_TOTAL // NUM_WORKERS
CHUNK = 512
N_CHUNKS = B_PER_W // CHUNK
NBUF = 4


def _emb_kernel(idx_hbm, table_hbm, out_hbm, idx_v, rows_v, isem, gsem, ssem):
    wid = lax.axis_index("s") * 2 + lax.axis_index("c")
    base = wid * B_PER_W

    def idx_start(i, b):
        pltpu.async_copy(
            idx_hbm.at[pl.ds(base + i * CHUNK, CHUNK)], idx_v.at[b], isem.at[b]
        )

    def idx_wait(b):
        pltpu.make_async_copy(
            idx_hbm.at[pl.ds(base, CHUNK)], idx_v.at[b], isem.at[b]
        ).wait()

    def gather_start(b):
        pltpu.async_copy(table_hbm.at[idx_v.at[b]], rows_v.at[b], gsem.at[b])

    def gather_wait(b):
        pltpu.make_async_copy(
            table_hbm.at[idx_v.at[b]], rows_v.at[b], gsem.at[b]
        ).wait()

    def store_start(i, b):
        pltpu.async_copy(
            rows_v.at[b], out_hbm.at[pl.ds(base + i * CHUNK, CHUNK)], ssem.at[b]
        )

    def store_wait(b):
        pltpu.make_async_copy(
            rows_v.at[b], out_hbm.at[pl.ds(base, CHUNK)], ssem.at[b]
        ).wait()

    for b in range(NBUF):
        idx_start(b, b)

    for b in range(NBUF):
        idx_wait(b)
        gather_start(b)
        if b >= 1:
            b1 = b - 1
            gather_wait(b1)
            store_start(b - 1, b1)
            idx_start(b - 1 + NBUF, b1)

    @pl.loop(NBUF, N_CHUNKS - NBUF, step=NBUF)
    def _steady(g):
        for b in range(NBUF):
            i = g + b
            store_wait(b)
            idx_wait(b)
            gather_start(b)
            b1 = (b - 1) % NBUF
            gather_wait(b1)
            store_start(i - 1, b1)
            idx_start(i - 1 + NBUF, b1)

    for b in range(NBUF):
        i = N_CHUNKS - NBUF + b
        store_wait(b)
        idx_wait(b)
        gather_start(b)
        b1 = (b - 1) % NBUF
        gather_wait(b1)
        store_start(i - 1, b1)
        if i - 1 + NBUF < N_CHUNKS:
            idx_start(i - 1 + NBUF, b1)

    bl = (N_CHUNKS - 1) % NBUF
    gather_wait(bl)
    store_start(N_CHUNKS - 1, bl)
    for b in range(NBUF):
        store_wait(b)


@jax.jit
def _embedding_lookup(x_flat, table):
    mesh = plsc.VectorSubcoreMesh(core_axis_name="c", subcore_axis_name="s")
    k = functools.partial(
        pl.kernel,
        out_type=jax.ShapeDtypeStruct((B_TOTAL, EMB_DIM), jnp.float32),
        mesh=mesh,
        scratch_types=[
            pltpu.VMEM((NBUF, CHUNK), jnp.int32),
            pltpu.VMEM((NBUF, CHUNK, EMB_DIM), jnp.float32),
            pltpu.SemaphoreType.DMA((NBUF,)),
            pltpu.SemaphoreType.DMA((NBUF,)),
            pltpu.SemaphoreType.DMA((NBUF,)),
        ],
        compiler_params=pltpu.CompilerParams(use_tc_tiling_on_sc=False),
    )(_emb_kernel)
    return k(x_flat, table)


def kernel(x, table):
    out = _embedding_lookup(x.reshape(-1).astype(jnp.int32), table)
    return out.reshape(BATCH, HIST, EMB_DIM)


# speedup vs baseline: 5.0441x; 1.0005x over previous
import functools

import jax
import jax.numpy as jnp
from jax import lax
from jax.experimental import pallas as pl
from jax.experimental.pallas import tpu as pltpu
from jax.experimental.pallas import tpu_sc as plsc

BATCH = 16384
HIST = 200
EMB_DIM = 32

NUM_WORKERS = 32
B_TOTAL = BATCH * HIST
B_PER_W = B_TOTAL // NUM_WORKERS
CHUNK = 512
N_CHUNKS = B_PER_W // CHUNK
NBUF = 5
K = 3
assert N_CHUNKS % NBUF == 0 and 0 < K < NBUF


def _emb_kernel(idx_hbm, table_hbm, out_hbm, idx_v, rows_v, isem, gsem, ssem):
    wid = lax.axis_index("s") * 2 + lax.axis_index("c")
    base = wid * B_PER_W

    def idx_start(i, b):
        pltpu.async_copy(
            idx_hbm.at[pl.ds(base + i * CHUNK, CHUNK)], idx_v.at[b], isem.at[b]
        )

    def idx_wait(b):
        pltpu.make_async_copy(
            idx_hbm.at[pl.ds(base, CHUNK)], idx_v.at[b], isem.at[b]
        ).wait()

    def gather_start(b):
        pltpu.async_copy(table_hbm.at[idx_v.at[b]], rows_v.at[b], gsem.at[b])

    def gather_wait(b):
        pltpu.make_async_copy(
            table_hbm.at[idx_v.at[b]], rows_v.at[b], gsem.at[b]
        ).wait()

    def store_start(i, b):
        pltpu.async_copy(
            rows_v.at[b], out_hbm.at[pl.ds(base + i * CHUNK, CHUNK)], ssem.at[b]
        )

    def store_wait(b):
        pltpu.make_async_copy(
            rows_v.at[b], out_hbm.at[pl.ds(base, CHUNK)], ssem.at[b]
        ).wait()

    def finish(j, b1, prefetch):
        gather_wait(b1)
        store_start(j, b1)
        if prefetch:
            idx_start(j + NBUF, b1)

    for b in range(NBUF):
        idx_start(b, b)

    for b in range(NBUF):
        idx_wait(b)
        gather_start(b)
        j = b - K
        if j >= 0:
            finish(j, j % NBUF, prefetch=True)

    @pl.loop(NBUF, N_CHUNKS - NBUF, step=NBUF)
    def _steady(g):
        for b in range(NBUF):
            i = g + b
            store_wait(b)
            idx_wait(b)
            gather_start(b)
            finish(i - K, (b - K) % NBUF, prefetch=True)

    for b in range(NBUF):
        i = N_CHUNKS - NBUF + b
        store_wait(b)
        idx_wait(b)
        gather_start(b)
        j = i - K
        finish(j, j % NBUF, prefetch=j + NBUF < N_CHUNKS)

    for j in range(N_CHUNKS - K, N_CHUNKS):
        finish(j, j % NBUF, prefetch=False)
    for b in range(NBUF):
        store_wait(b)


@jax.jit
def _embedding_lookup(x_flat, table):
    mesh = plsc.VectorSubcoreMesh(core_axis_name="c", subcore_axis_name="s")
    k = functools.partial(
        pl.kernel,
        out_type=jax.ShapeDtypeStruct((B_TOTAL, EMB_DIM), jnp.float32),
        mesh=mesh,
        scratch_types=[
            pltpu.VMEM((NBUF, CHUNK), jnp.int32),
            pltpu.VMEM((NBUF, CHUNK, EMB_DIM), jnp.float32),
            pltpu.SemaphoreType.DMA((NBUF,)),
            pltpu.SemaphoreType.DMA((NBUF,)),
            pltpu.SemaphoreType.DMA((NBUF,)),
        ],
        compiler_params=pltpu.CompilerParams(use_tc_tiling_on_sc=False),
    )(_emb_kernel)
    return k(x_flat, table)


def kernel(x, table):
    out = _embedding_lookup(x.reshape(-1).astype(jnp.int32), table)
    return out.reshape(BATCH, HIST, EMB_DIM)


# speedup vs baseline: 5.0468x; 1.0005x over previous
import functools

import jax
import jax.numpy as jnp
from jax import lax
from jax.experimental import pallas as pl
from jax.experimental.pallas import tpu as pltpu
from jax.experimental.pallas import tpu_sc as plsc

BATCH = 16384
HIST = 200
EMB_DIM = 32

NUM_WORKERS = 32
B_TOTAL = BATCH * HIST
B_PER_W = B_TOTAL // NUM_WORKERS
ROWS_PER_CHUNK = 4
CHUNK = ROWS_PER_CHUNK * HIST
N_CHUNKS = B_PER_W // CHUNK
NBUF = 4
K = 3
assert N_CHUNKS % NBUF == 0 and 0 < K < NBUF


def _emb_kernel(idx_hbm, table_hbm, out_hbm, idx_v, rows_v, isem, gsem, ssem):
    wid = lax.axis_index("s") * 2 + lax.axis_index("c")
    base = wid * B_PER_W

    def idx_start(i, b):
        pltpu.async_copy(
            idx_hbm.at[pl.ds(base + i * CHUNK, CHUNK)], idx_v.at[b], isem.at[b]
        )

    def idx_wait(b):
        pltpu.make_async_copy(
            idx_hbm.at[pl.ds(base, CHUNK)], idx_v.at[b], isem.at[b]
        ).wait()

    def gather_start(b):
        pltpu.async_copy(table_hbm.at[idx_v.at[b]], rows_v.at[b], gsem.at[b])

    def gather_wait(b):
        pltpu.make_async_copy(
            table_hbm.at[idx_v.at[b]], rows_v.at[b], gsem.at[b]
        ).wait()

    row_base = wid * (B_PER_W // HIST)

    def store_start(i, b):
        for r in range(ROWS_PER_CHUNK):
            pltpu.async_copy(
                rows_v.at[b].at[pl.ds(r * HIST, HIST)],
                out_hbm.at[row_base + i * ROWS_PER_CHUNK + r],
                ssem.at[b],
            )

    def store_wait(b):
        for _ in range(ROWS_PER_CHUNK):
            pltpu.make_async_copy(
                rows_v.at[b].at[pl.ds(0, HIST)],
                out_hbm.at[row_base],
                ssem.at[b],
            ).wait()

    def finish(j, b1, prefetch):
        gather_wait(b1)
        store_start(j, b1)
        if prefetch:
            idx_start(j + NBUF, b1)

    for b in range(NBUF):
        idx_start(b, b)

    for b in range(NBUF):
        idx_wait(b)
        gather_start(b)
        j = b - K
        if j >= 0:
            finish(j, j % NBUF, prefetch=True)

    @pl.loop(NBUF, N_CHUNKS - NBUF, step=NBUF)
    def _steady(g):
        for b in range(NBUF):
            i = g + b
            store_wait(b)
            idx_wait(b)
            gather_start(b)
            finish(i - K, (b - K) % NBUF, prefetch=True)

    for b in range(NBUF):
        i = N_CHUNKS - NBUF + b
        store_wait(b)
        idx_wait(b)
        gather_start(b)
        j = i - K
        finish(j, j % NBUF, prefetch=j + NBUF < N_CHUNKS)

    for j in range(N_CHUNKS - K, N_CHUNKS):
        finish(j, j % NBUF, prefetch=False)
    for b in range(NBUF):
        store_wait(b)


@jax.jit
def _embedding_lookup(x_flat, table):
    mesh = plsc.VectorSubcoreMesh(core_axis_name="c", subcore_axis_name="s")
    k = functools.partial(
        pl.kernel,
        out_type=jax.ShapeDtypeStruct((BATCH, HIST, EMB_DIM), jnp.float32),
        mesh=mesh,
        scratch_types=[
            pltpu.VMEM((NBUF, CHUNK), jnp.int32),
            pltpu.VMEM((NBUF, CHUNK, EMB_DIM), jnp.float32),
            pltpu.SemaphoreType.DMA((NBUF,)),
            pltpu.SemaphoreType.DMA((NBUF,)),
            pltpu.SemaphoreType.DMA((NBUF,)),
        ],
        compiler_params=pltpu.CompilerParams(use_tc_tiling_on_sc=False),
    )(_emb_kernel)
    return k(x_flat, table)


def kernel(x, table):
    return _embedding_lookup(x.reshape(-1).astype(jnp.int32), table)


# speedup vs baseline: 5.5344x; 1.0966x over previous
import functools

import jax
import jax.numpy as jnp
from jax import lax
from jax.experimental import pallas as pl
from jax.experimental.pallas import tpu as pltpu
from jax.experimental.pallas import tpu_sc as plsc

BATCH = 16384
HIST = 200
EMB_DIM = 32

NUM_WORKERS = 32
B_TOTAL = BATCH * HIST
B_PER_W = BATCH // NUM_WORKERS
CHUNK = B_PER_W
N_CHUNKS = HIST
NBUF = 5
K = 3
assert N_CHUNKS % NBUF == 0 and 0 < K < NBUF


def _emb_kernel(idx_hbm, table_hbm, out_hbm, idx_v, rows_v, isem, gsem, ssem):
    wid = lax.axis_index("s") * 2 + lax.axis_index("c")
    b0 = wid * B_PER_W

    def idx_start(i, b):
        pltpu.async_copy(
            idx_hbm.at[pl.ds(i * BATCH + b0, CHUNK)], idx_v.at[b], isem.at[b]
        )

    def idx_wait(b):
        pltpu.make_async_copy(
            idx_hbm.at[pl.ds(b0, CHUNK)], idx_v.at[b], isem.at[b]
        ).wait()

    def gather_start(b):
        pltpu.async_copy(table_hbm.at[idx_v.at[b]], rows_v.at[b], gsem.at[b])

    def gather_wait(b):
        pltpu.make_async_copy(
            table_hbm.at[idx_v.at[b]], rows_v.at[b], gsem.at[b]
        ).wait()

    def store_start(i, b):
        pltpu.async_copy(
            rows_v.at[b], out_hbm.at[i, pl.ds(b0, CHUNK)], ssem.at[b]
        )

    def store_wait(b):
        pltpu.make_async_copy(
            rows_v.at[b], out_hbm.at[0, pl.ds(b0, CHUNK)], ssem.at[b]
        ).wait()

    def finish(j, b1, prefetch):
        gather_wait(b1)
        store_start(j, b1)
        if prefetch:
            idx_start(j + NBUF, b1)

    for b in range(NBUF):
        idx_start(b, b)

    for b in range(NBUF):
        idx_wait(b)
        gather_start(b)
        j = b - K
        if j >= 0:
            finish(j, j % NBUF, prefetch=True)

    @pl.loop(NBUF, N_CHUNKS - NBUF, step=NBUF)
    def _steady(g):
        for b in range(NBUF):
            i = g + b
            store_wait(b)
            idx_wait(b)
            gather_start(b)
            finish(i - K, (b - K) % NBUF, prefetch=True)

    for b in range(NBUF):
        i = N_CHUNKS - NBUF + b
        store_wait(b)
        idx_wait(b)
        gather_start(b)
        j = i - K
        finish(j, j % NBUF, prefetch=j + NBUF < N_CHUNKS)

    for j in range(N_CHUNKS - K, N_CHUNKS):
        finish(j, j % NBUF, prefetch=False)
    for b in range(NBUF):
        store_wait(b)


@jax.jit
def _embedding_lookup(xt_flat, table):
    mesh = plsc.VectorSubcoreMesh(core_axis_name="c", subcore_axis_name="s")
    k = functools.partial(
        pl.kernel,
        out_type=jax.ShapeDtypeStruct((HIST, BATCH, EMB_DIM), jnp.float32),
        mesh=mesh,
        scratch_types=[
            pltpu.VMEM((NBUF, CHUNK), jnp.int32),
            pltpu.VMEM((NBUF, CHUNK, EMB_DIM), jnp.float32),
            pltpu.SemaphoreType.DMA((NBUF,)),
            pltpu.SemaphoreType.DMA((NBUF,)),
            pltpu.SemaphoreType.DMA((NBUF,)),
        ],
        compiler_params=pltpu.CompilerParams(use_tc_tiling_on_sc=False),
    )(_emb_kernel)
    return k(xt_flat, table)


def kernel(x, table):
    xt_flat = x.T.reshape(-1).astype(jnp.int32)
    out_t = _embedding_lookup(xt_flat, table)
    return jnp.transpose(out_t, (1, 0, 2))
